# Initial kernel scaffold; baseline (speedup 1.0000x reference)
import functools

import jax
import jax.numpy as jnp
import numpy as np
from jax import lax
from jax.experimental import pallas as pl
from jax.experimental.pallas import tpu as pltpu
from jax.experimental.pallas import tpu_sc as plsc

N = 4096
B = 1024
L = 50
LL = L * L
LL_PAD = 2560
CHUNKS = LL_PAD // 16
DMA_ROWS = LL_PAD // 128

_info = plsc.get_sparse_core_info()
NC, NS = _info.num_cores, _info.num_subcores
NW = NC * NS
B_PER_W = B // NW

_k = np.arange(LL_PAD)
_i_sel = np.where(_k < LL, _k // L, 0).astype(np.int32).reshape(CHUNKS, 16)
_j_sel = np.where(_k < LL, _k % L, 0).astype(np.int32).reshape(CHUNKS, 16)


def _sc_kernel(table_hbm, idx_hbm, out_hbm, ind_v, flat_v, vals_v, sem):
    wid = lax.axis_index("s") * NC + lax.axis_index("c")
    base = wid * B_PER_W

    def body(t, carry):
        b = base + t
        pltpu.sync_copy(idx_hbm.at[b], ind_v)
        for c in range(CHUNKS):
            rs = jnp.asarray(_i_sel[c], dtype=jnp.int32)
            cs = jnp.asarray(_j_sel[c], dtype=jnp.int32)
            rv = plsc.load_gather(ind_v, [rs])
            cv = plsc.load_gather(ind_v, [cs])
            flat = rv * N + cv
            flat_v[c // 8, pl.ds((c % 8) * 16, 16)] = flat
        copies = []
        for r in range(DMA_ROWS):
            copies.append(
                pltpu.async_copy(
                    table_hbm.at[flat_v.at[r]],
                    vals_v.at[pl.ds(r * 128, 128)],
                    sem,
                )
            )
        for cp in copies:
            cp.wait()
        pltpu.sync_copy(vals_v, out_hbm.at[b])
        return carry

    lax.fori_loop(0, B_PER_W, body, 0)


@jax.jit
def _run(table_flat, idx32):
    mesh = plsc.VectorSubcoreMesh(core_axis_name="c", subcore_axis_name="s")
    k = functools.partial(
        pl.kernel,
        mesh=mesh,
        out_type=jax.ShapeDtypeStruct((B, LL_PAD), jnp.float32),
        scratch_types=[
            pltpu.VMEM((L,), jnp.int32),
            pltpu.VMEM((DMA_ROWS, 128), jnp.int32),
            pltpu.VMEM((LL_PAD,), jnp.float32),
            pltpu.SemaphoreType.DMA,
        ],
    )(_sc_kernel)
    return k(table_flat, idx32)


def kernel(innerVars, indices):
    table_flat = innerVars.reshape(-1)
    idx32 = indices.astype(jnp.int32)
    out = _run(table_flat, idx32)
    return out[:, :LL].reshape(B, L, L)


# speedup vs baseline: 6.9707x; 6.9707x over previous
import functools

import jax
import jax.numpy as jnp
import numpy as np
from jax import lax
from jax.experimental import pallas as pl
from jax.experimental.pallas import tpu as pltpu
from jax.experimental.pallas import tpu_sc as plsc

N = 4096
B = 1024
L = 50
LL = L * L
LL_PAD = 2560
CHUNKS = LL_PAD // 16
DMA_ROWS = LL_PAD // 128

_info = plsc.get_sparse_core_info()
NC, NS = _info.num_cores, _info.num_subcores
NW = NC * NS
B_PER_W = B // NW

_k = np.arange(LL_PAD)
_i_sel = np.where(_k < LL, _k // L, 0).astype(np.int32)
_j_sel = np.where(_k < LL, _k % L, 0).astype(np.int32)


def _sc_kernel(table_hbm, idx_hbm, isel_hbm, jsel_hbm, out_hbm,
               ind_v, isel_v, jsel_v, flat_v, vals_v, sem):
    wid = lax.axis_index("s") * NC + lax.axis_index("c")
    base = wid * B_PER_W
    pltpu.sync_copy(isel_hbm, isel_v)
    pltpu.sync_copy(jsel_hbm, jsel_v)

    def body(t, carry):
        b = base + t
        pltpu.sync_copy(idx_hbm.at[b], ind_v)
        for c in range(CHUNKS):
            rs = isel_v[pl.ds(c * 16, 16)]
            cs = jsel_v[pl.ds(c * 16, 16)]
            rv = plsc.load_gather(ind_v, [rs])
            cv = plsc.load_gather(ind_v, [cs])
            flat = rv * N + cv
            flat_v[c // 8, pl.ds((c % 8) * 16, 16)] = flat
        copies = []
        for r in range(DMA_ROWS):
            copies.append(
                pltpu.async_copy(
                    table_hbm.at[flat_v.at[r]],
                    vals_v.at[pl.ds(r * 128, 128)],
                    sem,
                )
            )
        for cp in copies:
            cp.wait()
        pltpu.sync_copy(vals_v, out_hbm.at[b])
        return carry

    lax.fori_loop(0, B_PER_W, body, 0)


@jax.jit
def _run(table_flat, idx32, isel, jsel):
    mesh = plsc.VectorSubcoreMesh(core_axis_name="c", subcore_axis_name="s")
    k = functools.partial(
        pl.kernel,
        mesh=mesh,
        compiler_params=pltpu.CompilerParams(needs_layout_passes=False),
        out_type=jax.ShapeDtypeStruct((B, LL_PAD), jnp.float32),
        scratch_types=[
            pltpu.VMEM((L,), jnp.int32),
            pltpu.VMEM((LL_PAD,), jnp.int32),
            pltpu.VMEM((LL_PAD,), jnp.int32),
            pltpu.VMEM((DMA_ROWS, 128), jnp.int32),
            pltpu.VMEM((LL_PAD,), jnp.float32),
            pltpu.SemaphoreType.DMA,
        ],
    )(_sc_kernel)
    return k(table_flat, idx32, isel, jsel)


def kernel(innerVars, indices):
    table_flat = innerVars.reshape(-1)
    idx32 = indices.astype(jnp.int32)
    out = _run(table_flat, idx32, jnp.asarray(_i_sel), jnp.asarray(_j_sel))
    return out[:, :LL].reshape(B, L, L)


# speedup vs baseline: 7.3662x; 1.0567x over previous
import functools

import jax
import jax.numpy as jnp
import numpy as np
from jax import lax
from jax.experimental import pallas as pl
from jax.experimental.pallas import tpu as pltpu
from jax.experimental.pallas import tpu_sc as plsc

N = 4096
B = 1024
L = 50
LL = L * L
LL_PAD = 2560
CHUNKS = LL_PAD // 16
DMA_ROWS = LL_PAD // 128

_info = plsc.get_sparse_core_info()
NC, NS = _info.num_cores, _info.num_subcores
NW = NC * NS
B_PER_W = B // NW

_k = np.arange(LL_PAD)
_i_sel = np.where(_k < LL, _k // L, 0).astype(np.int32)
_j_sel = np.where(_k < LL, _k % L, 0).astype(np.int32)


def _sc_kernel(table_hbm, idx_hbm, isel_hbm, jsel_hbm, out_hbm,
               ind_v, isel_v, jsel_v, flat_a, flat_b, vals_v, sem):
    wid = lax.axis_index("s") * NC + lax.axis_index("c")
    base = wid * B_PER_W
    pltpu.sync_copy(isel_hbm, isel_v)
    pltpu.sync_copy(jsel_hbm, jsel_v)
    pltpu.sync_copy(idx_hbm.at[pl.ds(base, B_PER_W)], ind_v)

    def compute_flat(t, flat_ref):
        tv = jnp.full((16,), 0, jnp.int32) + t
        for c in range(CHUNKS):
            rs = isel_v[pl.ds(c * 16, 16)]
            cs = jsel_v[pl.ds(c * 16, 16)]
            rv = plsc.load_gather(ind_v, [tv, rs])
            cv = plsc.load_gather(ind_v, [tv, cs])
            flat = rv * N + cv
            flat_ref[c // 8, pl.ds((c % 8) * 16, 16)] = flat

    def pair_body(u, carry):
        t0 = 2 * u
        t1 = t0 + 1
        compute_flat(t0, flat_a)
        cps_a = [
            pltpu.async_copy(
                table_hbm.at[flat_a.at[r]],
                vals_v.at[t0, pl.ds(r * 128, 128)], sem)
            for r in range(DMA_ROWS)
        ]
        compute_flat(t1, flat_b)
        cps_b = [
            pltpu.async_copy(
                table_hbm.at[flat_b.at[r]],
                vals_v.at[t1, pl.ds(r * 128, 128)], sem)
            for r in range(DMA_ROWS)
        ]
        for cp in cps_a + cps_b:
            cp.wait()
        return carry

    lax.fori_loop(0, B_PER_W // 2, pair_body, 0)
    pltpu.sync_copy(vals_v, out_hbm.at[pl.ds(base, B_PER_W)])


@jax.jit
def _run(table_flat, idx32, isel, jsel):
    mesh = plsc.VectorSubcoreMesh(core_axis_name="c", subcore_axis_name="s")
    k = functools.partial(
        pl.kernel,
        mesh=mesh,
        compiler_params=pltpu.CompilerParams(needs_layout_passes=False),
        out_type=jax.ShapeDtypeStruct((B, LL_PAD), jnp.float32),
        scratch_types=[
            pltpu.VMEM((B_PER_W, L), jnp.int32),
            pltpu.VMEM((LL_PAD,), jnp.int32),
            pltpu.VMEM((LL_PAD,), jnp.int32),
            pltpu.VMEM((DMA_ROWS, 128), jnp.int32),
            pltpu.VMEM((DMA_ROWS, 128), jnp.int32),
            pltpu.VMEM((B_PER_W, LL_PAD), jnp.float32),
            pltpu.SemaphoreType.DMA,
        ],
    )(_sc_kernel)
    return k(table_flat, idx32, isel, jsel)


def kernel(innerVars, indices):
    table_flat = innerVars.reshape(-1)
    idx32 = indices.astype(jnp.int32)
    out = _run(table_flat, idx32, jnp.asarray(_i_sel), jnp.asarray(_j_sel))
    return out[:, :LL].reshape(B, L, L)


# speedup vs baseline: 10.2202x; 1.3874x over previous
import functools

import jax
import jax.numpy as jnp
import numpy as np
from jax import lax
from jax.experimental import pallas as pl
from jax.experimental.pallas import tpu as pltpu
from jax.experimental.pallas import tpu_sc as plsc

N = 4096
B = 1024
L = 50
LP = 64
LL = L * L
LL_PAD = 2560
CHUNKS = LL_PAD // 16
DMA_ROWS = LL_PAD // 128

_info = plsc.get_sparse_core_info()
NC, NS = _info.num_cores, _info.num_subcores
NW = NC * NS
B_PER_W = B // NW

_k = np.arange(LL_PAD)
_i_sel = np.where(_k < LL, _k // L, 0).astype(np.int32)
_j_sel = np.where(_k < LL, _k % L, 0).astype(np.int32)


def _sc_kernel(table_hbm, idx_hbm, isel_hbm, jsel_hbm, out_hbm,
               ind_v, isel_v, jsel_v, rbase_v, cbase_v,
               flat_a, flat_b, vals_v, sem):
    wid = lax.axis_index("s") * NC + lax.axis_index("c")
    base = wid * B_PER_W
    pltpu.sync_copy(isel_hbm, isel_v)
    pltpu.sync_copy(jsel_hbm, jsel_v)
    pltpu.sync_copy(idx_hbm.at[pl.ds(base, B_PER_W)], ind_v)

    def compute_flat(t, flat_ref):
        tv = jnp.full((16,), 0, jnp.int32) + t
        for c in range(LP // 16):
            lanes = jnp.minimum(lax.iota(jnp.int32, 16) + c * 16, L - 1)
            iv = plsc.load_gather(ind_v, [tv, lanes])
            rbase_v[pl.ds(c * 16, 16)] = (
                lax.shift_left(lax.shift_right_logical(iv, 3), 15)
                + lax.shift_left(lax.bitwise_and(iv, 7), 7))
            cbase_v[pl.ds(c * 16, 16)] = (
                lax.shift_left(lax.shift_right_logical(iv, 7), 10)
                + lax.bitwise_and(iv, 127))
        for c in range(CHUNKS):
            rs = isel_v[pl.ds(c * 16, 16)]
            cs = jsel_v[pl.ds(c * 16, 16)]
            rv = plsc.load_gather(rbase_v, [rs])
            cv = plsc.load_gather(cbase_v, [cs])
            flat_ref[c // 8, pl.ds((c % 8) * 16, 16)] = rv + cv

    def fire(flat_ref, t):
        return [
            pltpu.async_copy(
                table_hbm.at[flat_ref.at[r]],
                vals_v.at[t, pl.ds(r * 128, 128)], sem)
            for r in range(DMA_ROWS)
        ]

    def pair_body(u, carry):
        t0 = 2 * u
        t1 = t0 + 1
        compute_flat(t0, flat_a)
        cps_a = fire(flat_a, t0)
        compute_flat(t1, flat_b)
        cps_b = fire(flat_b, t1)
        for cp in cps_a + cps_b:
            cp.wait()
        return carry

    lax.fori_loop(0, B_PER_W // 2, pair_body, 0)
    pltpu.sync_copy(vals_v, out_hbm.at[pl.ds(base, B_PER_W)])


@jax.jit
def _run(table_flat, idx32, isel, jsel):
    mesh = plsc.VectorSubcoreMesh(core_axis_name="c", subcore_axis_name="s")
    k = functools.partial(
        pl.kernel,
        mesh=mesh,
        compiler_params=pltpu.CompilerParams(needs_layout_passes=False),
        out_type=jax.ShapeDtypeStruct((B, LL_PAD), jnp.float32),
        scratch_types=[
            pltpu.VMEM((B_PER_W, L), jnp.int32),
            pltpu.VMEM((LL_PAD,), jnp.int32),
            pltpu.VMEM((LL_PAD,), jnp.int32),
            pltpu.VMEM((LP,), jnp.int32),
            pltpu.VMEM((LP,), jnp.int32),
            pltpu.VMEM((DMA_ROWS, 128), jnp.int32),
            pltpu.VMEM((DMA_ROWS, 128), jnp.int32),
            pltpu.VMEM((B_PER_W, LL_PAD), jnp.float32),
            pltpu.SemaphoreType.DMA,
        ],
    )(_sc_kernel)
    return k(table_flat, idx32, isel, jsel)


def kernel(innerVars, indices):
    table_flat = (
        innerVars.reshape(N // 8, 8, N // 128, 128)
        .transpose(0, 2, 1, 3)
        .reshape(-1)
    )
    idx32 = indices.astype(jnp.int32)
    out = _run(table_flat, idx32, jnp.asarray(_i_sel), jnp.asarray(_j_sel))
    return out[:, :LL].reshape(B, L, L)
